# baseline (speedup 1.0000x reference)
import jax
import jax.numpy as jnp
from jax import lax
from jax.experimental import pallas as pl
from jax.experimental.pallas import tpu as pltpu
from jax.experimental.pallas import tpu_sc as plsc

EMB = 512
NV = EMB // 16
T_TOKENS = 32768
B_BAGS = 16
TYPES = 128
NC = 2
NS = 16
NW = NC * NS
TPW = T_TOKENS // NW
CHUNK = 32
NCHUNK = TPW // CHUNK
NBUF = 4
NOUTER = NCHUNK // NBUF


def _sc_body(emb_hbm, feat_hbm, off_hbm, out_hbm,
             idx_all, rows_bufs, acc_v, off_v, off_sm, sems):
    cid = lax.axis_index("c")
    sid = lax.axis_index("s")
    wid = sid * NC + cid
    base = wid * TPW

    idx_dma = pltpu.make_async_copy(
        feat_hbm.at[pl.ds(base, TPW)], idx_all, sems[0])
    idx_dma.start()
    pltpu.sync_copy(off_hbm, off_v)

    zvec = jnp.zeros((16,), jnp.float32)

    def zero_body(b, _):
        for j in range(NV):
            acc_v[b, pl.ds(j * 16, 16)] = zvec
        return 0

    lax.fori_loop(0, B_BAGS, zero_body, 0)

    ovec = off_v[...]
    lanes = lax.iota(jnp.int32, 16)
    for b in range(B_BAGS):
        off_sm[b] = lax.reduce_max(
            jnp.where(lanes == b, ovec, jnp.int32(0)), (0,))
    off_sm[B_BAGS] = jnp.int32(T_TOKENS)

    def gather(c, rows, sem):
        start = pl.multiple_of(c * CHUNK, CHUNK)
        return pltpu.make_async_copy(
            emb_hbm.at[idx_all.at[pl.ds(start, CHUNK)]], rows, sem)

    def accumulate(rows_v, c):
        tbase = base + c * CHUNK

        def bag_body(b, _):
            lo = jnp.clip(off_sm[b] - tbase, 0, CHUNK)
            hi = jnp.clip(off_sm[b + 1] - tbase, 0, CHUNK)

            @pl.when(hi > lo)
            def _run():
                def tok_body(t, regs):
                    return tuple(
                        regs[j] + rows_v[t, pl.ds(j * 16, 16)]
                        for j in range(NV)
                    )

                regs = lax.fori_loop(
                    lo, hi, tok_body,
                    tuple(jnp.zeros((16,), jnp.float32) for _ in range(NV)))
                for j in range(NV):
                    sl = pl.ds(j * 16, 16)
                    acc_v[b, sl] = acc_v[b, sl] + regs[j]
            return 0

        lax.fori_loop(0, B_BAGS, bag_body, 0)

    idx_dma.wait()
    for k in range(NBUF):
        gather(k, rows_bufs[k], sems[k]).start()

    def outer_body(p, _):
        c0 = NBUF * p
        for k in range(NBUF):
            gather(c0 + k, rows_bufs[k], sems[k]).wait()
            accumulate(rows_bufs[k], c0 + k)

            @pl.when(p < NOUTER - 1)
            def _prefetch():
                gather(c0 + k + NBUF, rows_bufs[k], sems[k]).start()
        return 0

    lax.fori_loop(0, NOUTER, outer_body, 0)
    pltpu.sync_copy(acc_v, out_hbm.at[wid])


def _make_sc_kernel():
    mesh = plsc.VectorSubcoreMesh(core_axis_name="c", subcore_axis_name="s")
    return pl.kernel(
        _sc_body,
        out_type=jax.ShapeDtypeStruct((NW, B_BAGS, EMB), jnp.float32),
        mesh=mesh,
        compiler_params=pltpu.CompilerParams(
            needs_layout_passes=False, use_tc_tiling_on_sc=False),
        scratch_types=[
            pltpu.VMEM((TPW,), jnp.int32),
            [pltpu.VMEM((CHUNK, EMB), jnp.float32) for _ in range(NBUF)],
            pltpu.VMEM((B_BAGS, EMB), jnp.float32),
            pltpu.VMEM((B_BAGS,), jnp.int32),
            pltpu.SMEM((B_BAGS + 1,), jnp.int32),
            [pltpu.SemaphoreType.DMA for _ in range(NBUF)],
        ],
    )


def _tc_body(part_ref, off_ref, lin_ref, out_ref):
    sums = jnp.sum(part_ref[...], axis=0)
    off = off_ref[...]
    nxt = jnp.concatenate(
        [off[:, 1:], jnp.full((1, 1), T_TOKENS, jnp.int32)], axis=1)
    counts = (nxt - off).astype(jnp.float32)
    mean = sums / jnp.maximum(counts, 1.0).reshape(B_BAGS, 1)
    out_ref[...] = lax.dot_general(
        mean, lin_ref[...], (((1,), (1,)), ((), ())),
        preferred_element_type=jnp.float32)


@jax.jit
def kernel(feature_seq, offset_seq, emb_weight, lin_weight):
    partials = _make_sc_kernel()(emb_weight, feature_seq, offset_seq)
    return pl.pallas_call(
        _tc_body,
        out_shape=jax.ShapeDtypeStruct((B_BAGS, TYPES), jnp.float32),
    )(partials, offset_seq.reshape(1, B_BAGS), lin_weight)


# speedup vs baseline: 3.5902x; 3.5902x over previous
import jax
import jax.numpy as jnp
from jax import lax
from jax.experimental import pallas as pl
from jax.experimental.pallas import tpu as pltpu
from jax.experimental.pallas import tpu_sc as plsc

EMB = 512
NV = EMB // 16
T_TOKENS = 32768
B_BAGS = 16
TYPES = 128
NC = 2
NS = 16
NW = NC * NS
TPW = T_TOKENS // NW
CHUNK = 32
NCHUNK = TPW // CHUNK
NBUF = 4
NOUTER = NCHUNK // NBUF


def _sc_body(emb_hbm, feat_hbm, off_hbm, out_hbm,
             idx_all, rows_bufs, acc_v, off_v, off_sm, sems):
    cid = lax.axis_index("c")
    sid = lax.axis_index("s")
    wid = sid * NC + cid
    base = wid * TPW

    idx_dma = pltpu.make_async_copy(
        feat_hbm.at[pl.ds(base, TPW)], idx_all, sems[0])
    idx_dma.start()
    pltpu.sync_copy(off_hbm, off_v)

    zvec = jnp.zeros((16,), jnp.float32)

    def zero_body(b, _):
        for j in range(NV):
            acc_v[b, pl.ds(j * 16, 16)] = zvec
        return 0

    lax.fori_loop(0, B_BAGS, zero_body, 0)

    ovec = off_v[...]
    lanes = lax.iota(jnp.int32, 16)
    for b in range(B_BAGS):
        off_sm[b] = lax.reduce_max(
            jnp.where(lanes == b, ovec, jnp.int32(0)), (0,))
    off_sm[B_BAGS] = jnp.int32(T_TOKENS)

    def gather(c, rows, sem):
        start = pl.multiple_of(c * CHUNK, CHUNK)
        return pltpu.make_async_copy(
            emb_hbm.at[idx_all.at[pl.ds(start, CHUNK)]], rows, sem)

    def accumulate(rows_v, c):
        tbase = base + c * CHUNK

        def bag_body(b, _):
            lo = jnp.clip(off_sm[b] - tbase, 0, CHUNK)
            hi = jnp.clip(off_sm[b + 1] - tbase, 0, CHUNK)

            @pl.when(hi > lo)
            def _run():
                def tok_body(t, regs):
                    return tuple(
                        regs[j] + rows_v[t, pl.ds(j * 16, 16)]
                        for j in range(NV)
                    )

                regs = lax.fori_loop(
                    lo, hi, tok_body,
                    tuple(jnp.zeros((16,), jnp.float32) for _ in range(NV)))
                for j in range(NV):
                    sl = pl.ds(j * 16, 16)
                    acc_v[b, sl] = acc_v[b, sl] + regs[j]
            return 0

        lax.fori_loop(0, B_BAGS, bag_body, 0)

    idx_dma.wait()
    for k in range(NBUF):
        gather(k, rows_bufs[k], sems[k]).start()

    def outer_body(p, _):
        c0 = NBUF * p
        for k in range(NBUF):
            gather(c0 + k, rows_bufs[k], sems[k]).wait()
            accumulate(rows_bufs[k], c0 + k)

            @pl.when(p < NOUTER - 1)
            def _prefetch():
                gather(c0 + k + NBUF, rows_bufs[k], sems[k]).start()
        return 0

    lax.fori_loop(0, NOUTER, outer_body, 0)
    pltpu.sync_copy(acc_v, out_hbm.at[wid])


def _make_sc_kernel():
    mesh = plsc.VectorSubcoreMesh(core_axis_name="c", subcore_axis_name="s")
    return pl.kernel(
        _sc_body,
        out_type=jax.ShapeDtypeStruct((NW, B_BAGS, EMB), jnp.float32),
        mesh=mesh,
        compiler_params=pltpu.CompilerParams(needs_layout_passes=False),
        scratch_types=[
            pltpu.VMEM((TPW,), jnp.int32),
            [pltpu.VMEM((CHUNK, EMB), jnp.float32) for _ in range(NBUF)],
            pltpu.VMEM((B_BAGS, EMB), jnp.float32),
            pltpu.VMEM((B_BAGS,), jnp.int32),
            pltpu.SMEM((B_BAGS + 1,), jnp.int32),
            [pltpu.SemaphoreType.DMA for _ in range(NBUF)],
        ],
    )


def _tc_body(part_ref, off_ref, lin_ref, out_ref):
    sums = jnp.sum(part_ref[...], axis=0)
    off = off_ref[...]
    nxt = jnp.concatenate(
        [off[:, 1:], jnp.full((1, 1), T_TOKENS, jnp.int32)], axis=1)
    counts = (nxt - off).astype(jnp.float32)
    mean = sums / jnp.maximum(counts, 1.0).reshape(B_BAGS, 1)
    out_ref[...] = lax.dot_general(
        mean, lin_ref[...], (((1,), (1,)), ((), ())),
        preferred_element_type=jnp.float32)


@jax.jit
def kernel(feature_seq, offset_seq, emb_weight, lin_weight):
    partials = _make_sc_kernel()(emb_weight, feature_seq, offset_seq)
    return pl.pallas_call(
        _tc_body,
        out_shape=jax.ShapeDtypeStruct((B_BAGS, TYPES), jnp.float32),
    )(partials, offset_seq.reshape(1, B_BAGS), lin_weight)
